# Initial kernel scaffold; baseline (speedup 1.0000x reference)
import jax
import jax.numpy as jnp
from jax.experimental import pallas as pl


def kernel(logits):
    raise NotImplementedError("write your pallas kernel here")


# speedup vs baseline: 2.3371x; 2.3371x over previous
import functools

import jax
import jax.numpy as jnp
from jax import lax
from jax.experimental import pallas as pl
from jax.experimental.pallas import tpu as pltpu
from jax.experimental.pallas import tpu_sc as plsc

ROWS = 128
N = 32768
TOPK = 10
NC = 2
NS = 16
L = 16
NW = NC * NS
RPW = ROWS // NW
SB = 256
NSB = N // SB
NBLK = N // L

_NEG = float("-inf")


def _row_topk(ts, iota16):
  sv = []
  for g in range(8):
    def sb_body(s_, ssm_g, g=g):
      base = (g * 16 + s_) * SB
      m = ts[pl.ds(base, L)]
      for j in range(1, SB // L):
        m = jnp.maximum(m, ts[pl.ds(base + j * L, L)])
      smax = jnp.max(m)
      return jnp.where(iota16 == s_, smax, ssm_g)
    sv.append(lax.fori_loop(0, 16, sb_body, jnp.full((L,), _NEG, jnp.float32)))

  winsb = jnp.zeros((L,), jnp.int32)
  for k in range(TOPK):
    m = sv[0]
    for g in range(1, 8):
      m = jnp.maximum(m, sv[g])
    gv = jnp.max(m)
    cidx = jnp.full((L,), NSB, jnp.int32)
    for g in range(8):
      cidx = jnp.minimum(cidx, jnp.where(sv[g] == gv, iota16 + g * 16, NSB))
    sstar = jnp.min(cidx)
    winsb = jnp.where(iota16 == k, sstar, winsb)
    for g in range(8):
      sv[g] = jnp.where(iota16 + g * 16 == sstar, _NEG, sv[g])

  bmv = []
  bidv = []
  for k in range(TOPK):
    s_k = jnp.max(jnp.where(iota16 == k, winsb, 0))
    base = s_k * SB
    bm = jnp.full((L,), _NEG, jnp.float32)
    for j in range(SB // L):
      v = ts[pl.ds(base + j * L, L)]
      bm = jnp.where(iota16 == j, jnp.max(v), bm)
    bmv.append(bm)
    bidv.append(s_k * (SB // L) + iota16)

  winblk = jnp.zeros((L,), jnp.int32)
  for k in range(TOPK):
    m = bmv[0]
    for t in range(1, TOPK):
      m = jnp.maximum(m, bmv[t])
    gv = jnp.max(m)
    cidx = jnp.full((L,), NBLK, jnp.int32)
    for t in range(TOPK):
      cidx = jnp.minimum(cidx, jnp.where(bmv[t] == gv, bidv[t], NBLK))
    bstar = jnp.min(cidx)
    winblk = jnp.where(iota16 == k, bstar, winblk)
    for t in range(TOPK):
      bmv[t] = jnp.where(bidv[t] == bstar, _NEG, bmv[t])

  cv = []
  ci = []
  for k in range(TOPK):
    b_k = jnp.max(jnp.where(iota16 == k, winblk, 0))
    cv.append(ts[pl.ds(b_k * L, L)])
    ci.append(b_k * L + iota16)
  ov = jnp.zeros((L,), jnp.float32)
  oi = jnp.zeros((L,), jnp.int32)
  for k in range(TOPK):
    m = cv[0]
    for t in range(1, TOPK):
      m = jnp.maximum(m, cv[t])
    gv = jnp.max(m)
    cidx = jnp.full((L,), N, jnp.int32)
    for t in range(TOPK):
      cidx = jnp.minimum(cidx, jnp.where(cv[t] == gv, ci[t], N))
    istar = jnp.min(cidx)
    ov = jnp.where(iota16 == k, gv, ov)
    oi = jnp.where(iota16 == k, istar, oi)
    for t in range(TOPK):
      cv[t] = jnp.where(ci[t] == istar, _NEG, cv[t])
  return ov, oi


@functools.lru_cache(maxsize=1)
def _make_kernel():
  mesh = plsc.VectorSubcoreMesh(
      core_axis_name="c", subcore_axis_name="s",
      num_cores=NC, num_subcores=NS)

  @functools.partial(
      pl.kernel,
      out_type=[
          jax.ShapeDtypeStruct((ROWS, L), jnp.float32),
          jax.ShapeDtypeStruct((ROWS, L), jnp.int32),
      ],
      mesh=mesh,
      scratch_types=[
          pltpu.VMEM((N,), jnp.float32),
          pltpu.VMEM((L,), jnp.float32),
          pltpu.VMEM((L,), jnp.int32),
      ],
      compiler_params=pltpu.CompilerParams(needs_layout_passes=False),
  )
  def topk_kernel(logits_hbm, vals_hbm, idxs_hbm, ts, ov_ref, oi_ref):
    wid = lax.axis_index("s") * NC + lax.axis_index("c")
    row0 = wid * RPW
    iota16 = lax.iota(jnp.int32, L)

    def row_body(r, _):
      row = row0 + r
      pltpu.sync_copy(logits_hbm.at[row], ts)
      ov, oi = _row_topk(ts, iota16)
      ov_ref[...] = ov
      oi_ref[...] = oi
      pltpu.sync_copy(ov_ref, vals_hbm.at[row])
      pltpu.sync_copy(oi_ref, idxs_hbm.at[row])
      return 0

    lax.fori_loop(0, RPW, row_body, 0)

  return topk_kernel


@jax.jit
def kernel(logits):
  vals, idxs = _make_kernel()(logits)
  return vals[:, :TOPK], idxs[:, :TOPK]


# speedup vs baseline: 2.4038x; 1.0285x over previous
import functools

import jax
import jax.numpy as jnp
from jax import lax
from jax.experimental import pallas as pl
from jax.experimental.pallas import tpu as pltpu
from jax.experimental.pallas import tpu_sc as plsc

ROWS = 128
N = 32768
TOPK = 10
NC = 2
NS = 16
L = 16
NW = NC * NS
RPW = ROWS // NW
SB = 256
NSB = N // SB
BPS = SB // L
NBLK = N // L

_NEG = float("-inf")


def _row_topk(ts, iota16):
  sv = []
  for g in range(8):
    def sb_body(s_, ssm_g, g=g):
      sa = 2 * s_
      sb_i = sa + 1
      base_a = (g * 16 + sa) * SB
      base_b = (g * 16 + sb_i) * SB
      ma = ts[pl.ds(base_a, L)]
      mb = ts[pl.ds(base_b, L)]
      for j in range(1, SB // L):
        ma = jnp.maximum(ma, ts[pl.ds(base_a + j * L, L)])
        mb = jnp.maximum(mb, ts[pl.ds(base_b + j * L, L)])
      ssm_g = jnp.where(iota16 == sa, jnp.max(ma), ssm_g)
      ssm_g = jnp.where(iota16 == sb_i, jnp.max(mb), ssm_g)
      return ssm_g
    sv.append(lax.fori_loop(0, 8, sb_body, jnp.full((L,), _NEG, jnp.float32)))

  winsb = jnp.zeros((L,), jnp.int32)
  for k in range(TOPK):
    m = sv[0]
    for g in range(1, 8):
      m = jnp.maximum(m, sv[g])
    gv = jnp.max(m)
    cidx = jnp.full((L,), NSB, jnp.int32)
    for g in range(8):
      cidx = jnp.minimum(cidx, jnp.where(sv[g] == gv, iota16 + g * 16, NSB))
    sstar = jnp.min(cidx)
    winsb = jnp.where(iota16 == k, sstar, winsb)
    for g in range(8):
      sv[g] = jnp.where(iota16 + g * 16 == sstar, _NEG, sv[g])

  bmv = []
  bidv = []
  for k in range(TOPK):
    s_k = jnp.max(jnp.where(iota16 == k, winsb, 0))
    gidx = s_k * SB + iota16 * L
    bm = plsc.load_gather(ts, [gidx])
    for j in range(1, L):
      bm = jnp.maximum(bm, plsc.load_gather(ts, [gidx + j]))
    bmv.append(bm)
    bidv.append(s_k * BPS + iota16)

  winblk = jnp.zeros((L,), jnp.int32)
  for k in range(TOPK):
    m = bmv[0]
    for t in range(1, TOPK):
      m = jnp.maximum(m, bmv[t])
    gv = jnp.max(m)
    cidx = jnp.full((L,), NBLK, jnp.int32)
    for t in range(TOPK):
      cidx = jnp.minimum(cidx, jnp.where(bmv[t] == gv, bidv[t], NBLK))
    bstar = jnp.min(cidx)
    winblk = jnp.where(iota16 == k, bstar, winblk)
    for t in range(TOPK):
      bmv[t] = jnp.where(bidv[t] == bstar, _NEG, bmv[t])

  cv = []
  ci = []
  for k in range(TOPK):
    b_k = jnp.max(jnp.where(iota16 == k, winblk, 0))
    cv.append(ts[pl.ds(b_k * L, L)])
    ci.append(b_k * L + iota16)
  ov = jnp.zeros((L,), jnp.float32)
  oi = jnp.zeros((L,), jnp.int32)
  for k in range(TOPK):
    m = cv[0]
    for t in range(1, TOPK):
      m = jnp.maximum(m, cv[t])
    gv = jnp.max(m)
    cidx = jnp.full((L,), N, jnp.int32)
    for t in range(TOPK):
      cidx = jnp.minimum(cidx, jnp.where(cv[t] == gv, ci[t], N))
    istar = jnp.min(cidx)
    ov = jnp.where(iota16 == k, gv, ov)
    oi = jnp.where(iota16 == k, istar, oi)
    for t in range(TOPK):
      cv[t] = jnp.where(ci[t] == istar, _NEG, cv[t])
  return ov, oi


@functools.lru_cache(maxsize=1)
def _make_kernel():
  mesh = plsc.VectorSubcoreMesh(
      core_axis_name="c", subcore_axis_name="s",
      num_cores=NC, num_subcores=NS)

  @functools.partial(
      pl.kernel,
      out_type=[
          jax.ShapeDtypeStruct((ROWS, L), jnp.float32),
          jax.ShapeDtypeStruct((ROWS, L), jnp.int32),
      ],
      mesh=mesh,
      scratch_types=[
          pltpu.VMEM((N,), jnp.float32),
          pltpu.VMEM((N,), jnp.float32),
          pltpu.VMEM((L,), jnp.float32),
          pltpu.VMEM((L,), jnp.int32),
          pltpu.SemaphoreType.DMA,
          pltpu.SemaphoreType.DMA,
      ],
      compiler_params=pltpu.CompilerParams(needs_layout_passes=False),
  )
  def topk_kernel(logits_hbm, vals_hbm, idxs_hbm, ts_a, ts_b, ov_ref, oi_ref,
                  sem_a, sem_b):
    wid = lax.axis_index("s") * NC + lax.axis_index("c")
    row0 = wid * RPW
    iota16 = lax.iota(jnp.int32, L)

    def emit(ts, row):
      ov, oi = _row_topk(ts, iota16)
      ov_ref[...] = ov
      oi_ref[...] = oi
      pltpu.sync_copy(ov_ref, vals_hbm.at[row])
      pltpu.sync_copy(oi_ref, idxs_hbm.at[row])

    pltpu.make_async_copy(logits_hbm.at[row0], ts_a, sem_a).start()

    def pair_body(i, _):
      ra = row0 + 2 * i
      rb = ra + 1
      pltpu.make_async_copy(logits_hbm.at[ra], ts_a, sem_a).wait()
      pltpu.make_async_copy(logits_hbm.at[rb], ts_b, sem_b).start()
      emit(ts_a, ra)
      pltpu.make_async_copy(logits_hbm.at[rb], ts_b, sem_b).wait()

      @pl.when(i + 1 < RPW // 2)
      def _():
        pltpu.make_async_copy(logits_hbm.at[ra + 2], ts_a, sem_a).start()

      emit(ts_b, rb)
      return 0

    lax.fori_loop(0, RPW // 2, pair_body, 0)

  return topk_kernel


@jax.jit
def kernel(logits):
  vals, idxs = _make_kernel()(logits)
  return vals[:, :TOPK], idxs[:, :TOPK]
